# baseline (speedup 1.0000x reference)
import functools

import jax
import jax.numpy as jnp
from jax import lax
from jax.experimental import pallas as pl
from jax.experimental.pallas import tpu as pltpu
from jax.experimental.pallas import tpu_sc as plsc

N = 10000
D = 128
K = 3
NC = 2
NS = 16
NW = NC * NS
N2 = 10240
RPT = N2 // NS
CB = 128
NCH = 84
NCHH = NCH // 2
EPW = NCH * CB
E2 = EPW * NW
BR = 320


def _mesh():
    return plsc.VectorSubcoreMesh(
        core_axis_name="c", subcore_axis_name="s",
        num_cores=NC, num_subcores=NS)


def _deg_body(e3, dp_out, ei_v, ones_v, zb_v, db_v, accd):
    cid = lax.axis_index("c")
    sid = lax.axis_index("s")
    wid = sid * NC + cid
    pltpu.sync_copy(e3.at[wid, pl.ds(0, NCH)], ei_v)
    z16 = jnp.zeros((16,), jnp.float32)
    o16 = jnp.ones((16,), jnp.float32)
    for i in range(CB // 16):
        ones_v[pl.ds(i * 16, 16)] = o16

    def fill(i, c):
        zb_v[pl.ds(i * 16, 16)] = z16
        return c

    lax.fori_loop(0, RPT // 16, fill, 0)
    pltpu.sync_copy(zb_v, accd.at[pl.ds(sid * RPT, RPT)])
    plsc.subcore_barrier()

    def step(j, c):
        pltpu.sync_copy(ones_v, accd.at[ei_v.at[j, 1]], add=True)
        return c

    lax.fori_loop(0, NCH, step, 0)
    plsc.subcore_barrier()
    pltpu.sync_copy(accd.at[pl.ds(sid * RPT, RPT)], db_v)
    pltpu.sync_copy(db_v, dp_out.at[cid, pl.ds(sid * RPT, RPT)])


_sc_deg = pl.kernel(
    _deg_body,
    out_type=jax.ShapeDtypeStruct((NC, N2), jnp.float32),
    mesh=_mesh(),
    scratch_types=[
        pltpu.VMEM((NCH, 2, CB), jnp.int32),
        pltpu.VMEM((CB,), jnp.float32),
        pltpu.VMEM((RPT,), jnp.float32),
        pltpu.VMEM((RPT,), jnp.float32),
        pltpu.VMEM_SHARED((N2,), jnp.float32),
    ],
)


def _hop_body(y_hbm, e3, p_out, ei_v, rows0, rows1, acc, gs0, gs1):
    cid = lax.axis_index("c")
    sid = lax.axis_index("s")
    wid = sid * NC + cid
    rows = (rows0, rows1)
    gs = (gs0, gs1)

    z16 = jnp.zeros((16,), jnp.float32)

    def zfill(r, c):
        for col in range(D // 16):
            rows0[r, pl.ds(col * 16, 16)] = z16
        return c

    lax.fori_loop(0, CB, zfill, 0)
    for b in range(RPT // CB):
        pltpu.sync_copy(rows0, acc.at[pl.ds(sid * RPT + b * CB, CB)])
    plsc.subcore_barrier()

    for h in range(2):
        pltpu.sync_copy(e3.at[wid, pl.ds(h * NCHH, NCHH)], ei_v)

        def step(j, c):
            pltpu.async_copy(y_hbm.at[ei_v.at[j, 0]], rows[0], gs[0]).wait()
            pltpu.sync_copy(rows[0], acc.at[ei_v.at[j, 1]], add=True)
            return c

        lax.fori_loop(0, NCHH, step, 0)
    plsc.subcore_barrier()
    for b in range(RPT // CB):
        base = sid * RPT + b * CB
        pltpu.sync_copy(acc.at[pl.ds(base, CB)], rows0)
        pltpu.sync_copy(rows0, p_out.at[cid, pl.ds(base, CB)])


_sc_hop = pl.kernel(
    _hop_body,
    out_type=jax.ShapeDtypeStruct((NC, N2, D), jnp.float32),
    mesh=_mesh(),
    scratch_types=[
        pltpu.VMEM((NCHH, 2, CB), jnp.int32),
        pltpu.VMEM((CB, D), jnp.float32),
        pltpu.VMEM((CB, D), jnp.float32),
        pltpu.VMEM_SHARED((N2, D), jnp.float32),
        pltpu.SemaphoreType.DMA,
        pltpu.SemaphoreType.DMA,
    ],
)


def _init_tc(dp_ref, x_ref, dinv_ref, y_ref):
    deg = dp_ref[0] + dp_ref[1]
    dinv = lax.rsqrt(jnp.maximum(deg, 1.0))
    dinv_ref[...] = dinv
    y_ref[...] = x_ref[...] * dinv


_tc_init = pl.pallas_call(
    _init_tc,
    grid=(N2 // BR,),
    in_specs=[
        pl.BlockSpec((NC, BR, 1), lambda i: (0, i, 0)),
        pl.BlockSpec((BR, D), lambda i: (i, 0)),
    ],
    out_specs=[
        pl.BlockSpec((BR, 1), lambda i: (i, 0)),
        pl.BlockSpec((BR, D), lambda i: (i, 0)),
    ],
    out_shape=[
        jax.ShapeDtypeStruct((N2, 1), jnp.float32),
        jax.ShapeDtypeStruct((N2, D), jnp.float32),
    ],
)


def _mid_tc(p_ref, s_ref, dinv_ref, s_out, y_out):
    dinv = dinv_ref[...]
    xn = (p_ref[0] + p_ref[1]) * dinv
    s_out[...] = s_ref[...] + xn
    y_out[...] = xn * dinv


_tc_mid = pl.pallas_call(
    _mid_tc,
    grid=(N2 // BR,),
    in_specs=[
        pl.BlockSpec((NC, BR, D), lambda i: (0, i, 0)),
        pl.BlockSpec((BR, D), lambda i: (i, 0)),
        pl.BlockSpec((BR, 1), lambda i: (i, 0)),
    ],
    out_specs=[
        pl.BlockSpec((BR, D), lambda i: (i, 0)),
        pl.BlockSpec((BR, D), lambda i: (i, 0)),
    ],
    out_shape=[
        jax.ShapeDtypeStruct((N2, D), jnp.float32),
        jax.ShapeDtypeStruct((N2, D), jnp.float32),
    ],
)


def _fin_tc(p_ref, s_ref, dinv_ref, s_out):
    xn = (p_ref[0] + p_ref[1]) * dinv_ref[...]
    s_out[...] = (s_ref[...] + xn) * (1.0 / (K + 1))


_tc_fin = pl.pallas_call(
    _fin_tc,
    grid=(N2 // BR,),
    in_specs=[
        pl.BlockSpec((NC, BR, D), lambda i: (0, i, 0)),
        pl.BlockSpec((BR, D), lambda i: (i, 0)),
        pl.BlockSpec((BR, 1), lambda i: (i, 0)),
    ],
    out_specs=pl.BlockSpec((BR, D), lambda i: (i, 0)),
    out_shape=jax.ShapeDtypeStruct((N2, D), jnp.float32),
)


def kernel(inputs, edge_index):
    src = edge_index[0].astype(jnp.int32)
    dst = edge_index[1].astype(jnp.int32)
    loop = jnp.arange(N, dtype=jnp.int32)
    pad = jnp.full((E2 - (src.shape[0] + N),), N, jnp.int32)
    src3 = jnp.concatenate([src, loop, pad]).reshape(NW, NCH, CB)
    dst3 = jnp.concatenate([dst, loop, pad]).reshape(NW, NCH, CB)
    e3 = jnp.stack([src3, dst3], axis=2)
    xpad = jnp.zeros((N2, D), jnp.float32).at[:N].set(inputs)

    dp = _sc_deg(e3)
    dinvc, y = _tc_init(dp.reshape(NC, N2, 1), xpad)
    s = xpad
    for k in range(K):
        p = _sc_hop(y, e3)
        if k < K - 1:
            s, y = _tc_mid(p, s, dinvc)
        else:
            s = _tc_fin(p, s, dinvc)
    return s[:N]


# speedup vs baseline: 1.0948x; 1.0948x over previous
import functools

import jax
import jax.numpy as jnp
from jax import lax
from jax.experimental import pallas as pl
from jax.experimental.pallas import tpu as pltpu
from jax.experimental.pallas import tpu_sc as plsc

N = 10000
D = 128
K = 3
NC = 2
NS = 16
NW = NC * NS
N2 = 10240
RPT = N2 // NS
CB = 128
NCH = 84
NCHH = NCH // 2
EPW = NCH * CB
E2 = EPW * NW
BR = 320


def _mesh():
    return plsc.VectorSubcoreMesh(
        core_axis_name="c", subcore_axis_name="s",
        num_cores=NC, num_subcores=NS)


def _deg_body(dst4, dp_out, ei_v, ones_v, zb_v, db_v, accd):
    cid = lax.axis_index("c")
    sid = lax.axis_index("s")
    wid = sid * NC + cid
    z16 = jnp.zeros((16,), jnp.float32)
    o16 = jnp.ones((16,), jnp.float32)
    for i in range(CB // 16):
        ones_v[pl.ds(i * 16, 16)] = o16

    def fill(i, c):
        zb_v[pl.ds(i * 16, 16)] = z16
        return c

    lax.fori_loop(0, RPT // 16, fill, 0)
    pltpu.sync_copy(zb_v, accd.at[pl.ds(sid * RPT, RPT)])
    plsc.subcore_barrier()

    for h in range(2):
        pltpu.sync_copy(dst4.at[wid, h], ei_v)

        def step(j, c):
            pltpu.sync_copy(ones_v, accd.at[ei_v.at[j]], add=True)
            return c

        lax.fori_loop(0, NCHH, step, 0)
    plsc.subcore_barrier()
    pltpu.sync_copy(accd.at[pl.ds(sid * RPT, RPT)], db_v)
    pltpu.sync_copy(db_v, dp_out.at[cid, pl.ds(sid * RPT, RPT)])


_sc_deg = pl.kernel(
    _deg_body,
    out_type=jax.ShapeDtypeStruct((NC, N2), jnp.float32),
    mesh=_mesh(),
    scratch_types=[
        pltpu.VMEM((NCHH, CB), jnp.int32),
        pltpu.VMEM((CB,), jnp.float32),
        pltpu.VMEM((RPT,), jnp.float32),
        pltpu.VMEM((RPT,), jnp.float32),
        pltpu.VMEM_SHARED((N2,), jnp.float32),
    ],
)


def _hop_body(y_hbm, src3, dst3, p_out, src_v, dst_v, rows0, rows1, acc,
              gs0, gs1):
    cid = lax.axis_index("c")
    sid = lax.axis_index("s")
    wid = sid * NC + cid
    rows = (rows0, rows1)
    gs = (gs0, gs1)

    z16 = jnp.zeros((16,), jnp.float32)

    def zfill(r, c):
        for col in range(D // 16):
            rows0[r, pl.ds(col * 16, 16)] = z16
        return c

    lax.fori_loop(0, CB, zfill, 0)
    for b in range(RPT // CB):
        pltpu.sync_copy(rows0, acc.at[pl.ds(sid * RPT + b * CB, CB)])
    plsc.subcore_barrier()

    for h in range(2):
        pltpu.sync_copy(src3.at[wid, h], src_v)
        pltpu.sync_copy(dst3.at[wid, h], dst_v)

        def step(j, c):
            pltpu.async_copy(y_hbm.at[src_v.at[j]], rows[0], gs[0]).wait()
            pltpu.sync_copy(rows[0], acc.at[dst_v.at[j]], add=True)
            return c

        lax.fori_loop(0, NCHH, step, 0)
    plsc.subcore_barrier()
    for b in range(RPT // CB):
        base = sid * RPT + b * CB
        pltpu.sync_copy(acc.at[pl.ds(base, CB)], rows0)
        pltpu.sync_copy(rows0, p_out.at[cid, pl.ds(base, CB)])


_sc_hop = pl.kernel(
    _hop_body,
    out_type=jax.ShapeDtypeStruct((NC, N2, D), jnp.float32),
    mesh=_mesh(),
    scratch_types=[
        pltpu.VMEM((NCHH, CB), jnp.int32),
        pltpu.VMEM((NCHH, CB), jnp.int32),
        pltpu.VMEM((CB, D), jnp.float32),
        pltpu.VMEM((CB, D), jnp.float32),
        pltpu.VMEM_SHARED((N2, D), jnp.float32),
        pltpu.SemaphoreType.DMA,
        pltpu.SemaphoreType.DMA,
    ],
)


def _init_tc(dp_ref, x_ref, dinv_ref, y_ref):
    deg = dp_ref[0] + dp_ref[1]
    dinv = lax.rsqrt(jnp.maximum(deg, 1.0))
    dinv_ref[...] = dinv
    y_ref[...] = x_ref[...] * dinv


_tc_init = pl.pallas_call(
    _init_tc,
    grid=(N2 // BR,),
    in_specs=[
        pl.BlockSpec((NC, BR, 1), lambda i: (0, i, 0)),
        pl.BlockSpec((BR, D), lambda i: (i, 0)),
    ],
    out_specs=[
        pl.BlockSpec((BR, 1), lambda i: (i, 0)),
        pl.BlockSpec((BR, D), lambda i: (i, 0)),
    ],
    out_shape=[
        jax.ShapeDtypeStruct((N2, 1), jnp.float32),
        jax.ShapeDtypeStruct((N2, D), jnp.float32),
    ],
)


def _mid_tc(p_ref, s_ref, dinv_ref, s_out, y_out):
    dinv = dinv_ref[...]
    xn = (p_ref[0] + p_ref[1]) * dinv
    s_out[...] = s_ref[...] + xn
    y_out[...] = xn * dinv


_tc_mid = pl.pallas_call(
    _mid_tc,
    grid=(N2 // BR,),
    in_specs=[
        pl.BlockSpec((NC, BR, D), lambda i: (0, i, 0)),
        pl.BlockSpec((BR, D), lambda i: (i, 0)),
        pl.BlockSpec((BR, 1), lambda i: (i, 0)),
    ],
    out_specs=[
        pl.BlockSpec((BR, D), lambda i: (i, 0)),
        pl.BlockSpec((BR, D), lambda i: (i, 0)),
    ],
    out_shape=[
        jax.ShapeDtypeStruct((N2, D), jnp.float32),
        jax.ShapeDtypeStruct((N2, D), jnp.float32),
    ],
)


def _fin_tc(p_ref, s_ref, dinv_ref, s_out):
    xn = (p_ref[0] + p_ref[1]) * dinv_ref[...]
    s_out[...] = (s_ref[...] + xn) * (1.0 / (K + 1))


_tc_fin = pl.pallas_call(
    _fin_tc,
    grid=(N2 // BR,),
    in_specs=[
        pl.BlockSpec((NC, BR, D), lambda i: (0, i, 0)),
        pl.BlockSpec((BR, D), lambda i: (i, 0)),
        pl.BlockSpec((BR, 1), lambda i: (i, 0)),
    ],
    out_specs=pl.BlockSpec((BR, D), lambda i: (i, 0)),
    out_shape=jax.ShapeDtypeStruct((N2, D), jnp.float32),
)


def kernel(inputs, edge_index):
    src = edge_index[0].astype(jnp.int32)
    dst = edge_index[1].astype(jnp.int32)
    loop = jnp.arange(N, dtype=jnp.int32)
    pad = jnp.full((E2 - (src.shape[0] + N),), N, jnp.int32)
    src3 = jnp.concatenate([src, loop, pad]).reshape(NW, 2, NCHH, CB)
    dst3 = jnp.concatenate([dst, loop, pad]).reshape(NW, 2, NCHH, CB)
    xpad = jnp.zeros((N2, D), jnp.float32).at[:N].set(inputs)

    dp = _sc_deg(dst3)
    dinvc, y = _tc_init(dp.reshape(NC, N2, 1), xpad)
    s = xpad
    for k in range(K):
        p = _sc_hop(y, src3, dst3)
        if k < K - 1:
            s, y = _tc_mid(p, s, dinvc)
        else:
            s = _tc_fin(p, s, dinvc)
    return s[:N]


# speedup vs baseline: 1.1046x; 1.0089x over previous
import functools

import jax
import jax.numpy as jnp
from jax import lax
from jax.experimental import pallas as pl
from jax.experimental.pallas import tpu as pltpu
from jax.experimental.pallas import tpu_sc as plsc

N = 10000
D = 128
K = 3
NC = 2
NS = 16
NW = NC * NS
N2 = 10240
RPT = N2 // NS
CB = 128
NCH = 84
NCHH = NCH // 2
EPW = NCH * CB
E2 = EPW * NW
BR = 320


def _mesh():
    return plsc.VectorSubcoreMesh(
        core_axis_name="c", subcore_axis_name="s",
        num_cores=NC, num_subcores=NS)


def _deg_body(dst4, dp_out, ei_v, ones_v, zb_v, db_v, accd):
    cid = lax.axis_index("c")
    sid = lax.axis_index("s")
    wid = sid * NC + cid
    z16 = jnp.zeros((16,), jnp.float32)
    o16 = jnp.ones((16,), jnp.float32)
    for i in range(CB // 16):
        ones_v[pl.ds(i * 16, 16)] = o16

    def fill(i, c):
        zb_v[pl.ds(i * 16, 16)] = z16
        return c

    lax.fori_loop(0, RPT // 16, fill, 0)
    pltpu.sync_copy(zb_v, accd.at[pl.ds(sid * RPT, RPT)])
    plsc.subcore_barrier()

    pltpu.sync_copy(dst4.at[wid], ei_v)

    def step(j, c):
        pltpu.sync_copy(ones_v, accd.at[ei_v.at[j]], add=True)
        return c

    lax.fori_loop(0, NCH, step, 0)
    plsc.subcore_barrier()
    pltpu.sync_copy(accd.at[pl.ds(sid * RPT, RPT)], db_v)
    pltpu.sync_copy(db_v, dp_out.at[cid, pl.ds(sid * RPT, RPT)])


_sc_deg = pl.kernel(
    _deg_body,
    out_type=jax.ShapeDtypeStruct((NC, N2), jnp.float32),
    mesh=_mesh(),
    scratch_types=[
        pltpu.VMEM((NCH, CB), jnp.int32),
        pltpu.VMEM((CB,), jnp.float32),
        pltpu.VMEM((RPT,), jnp.float32),
        pltpu.VMEM((RPT,), jnp.float32),
        pltpu.VMEM_SHARED((N2,), jnp.float32),
    ],
)


def _hop_body(y_hbm, src3, dst3, p_out, src_v, dst_v, rows_v, acc, sem):
    cid = lax.axis_index("c")
    sid = lax.axis_index("s")
    wid = sid * NC + cid
    pltpu.sync_copy(src3.at[wid], src_v)
    pltpu.sync_copy(dst3.at[wid], dst_v)
    z16 = jnp.zeros((16,), jnp.float32)

    def zfill(r, c):
        for col in range(D // 16):
            rows_v[r, pl.ds(col * 16, 16)] = z16
        return c

    lax.fori_loop(0, CB, zfill, 0)
    for b in range(RPT // CB):
        pltpu.sync_copy(rows_v, acc.at[pl.ds(sid * RPT + b * CB, CB)])
    plsc.subcore_barrier()

    def step(j, c):
        pltpu.async_copy(y_hbm.at[src_v.at[j]], rows_v, sem).wait()
        pltpu.sync_copy(rows_v, acc.at[dst_v.at[j]], add=True)
        return c

    lax.fori_loop(0, NCH, step, 0)
    plsc.subcore_barrier()
    for b in range(RPT // CB):
        base = sid * RPT + b * CB
        pltpu.sync_copy(acc.at[pl.ds(base, CB)], rows_v)
        pltpu.sync_copy(rows_v, p_out.at[cid, pl.ds(base, CB)])


_sc_hop = pl.kernel(
    _hop_body,
    out_type=jax.ShapeDtypeStruct((NC, N2, D), jnp.float32),
    mesh=_mesh(),
    scratch_types=[
        pltpu.VMEM((NCH, CB), jnp.int32),
        pltpu.VMEM((NCH, CB), jnp.int32),
        pltpu.VMEM((CB, D), jnp.float32),
        pltpu.VMEM_SHARED((N2, D), jnp.float32),
        pltpu.SemaphoreType.DMA,
    ],
)


def _init_tc(dp_ref, x_ref, dinv_ref, y_ref):
    deg = dp_ref[0] + dp_ref[1]
    dinv = lax.rsqrt(jnp.maximum(deg, 1.0))
    dinv_ref[...] = dinv
    y_ref[...] = x_ref[...] * dinv


_tc_init = pl.pallas_call(
    _init_tc,
    grid=(N2 // BR,),
    in_specs=[
        pl.BlockSpec((NC, BR, 1), lambda i: (0, i, 0)),
        pl.BlockSpec((BR, D), lambda i: (i, 0)),
    ],
    out_specs=[
        pl.BlockSpec((BR, 1), lambda i: (i, 0)),
        pl.BlockSpec((BR, D), lambda i: (i, 0)),
    ],
    out_shape=[
        jax.ShapeDtypeStruct((N2, 1), jnp.float32),
        jax.ShapeDtypeStruct((N2, D), jnp.float32),
    ],
)


def _mid_tc(p_ref, s_ref, dinv_ref, s_out, y_out):
    dinv = dinv_ref[...]
    xn = (p_ref[0] + p_ref[1]) * dinv
    s_out[...] = s_ref[...] + xn
    y_out[...] = xn * dinv


_tc_mid = pl.pallas_call(
    _mid_tc,
    grid=(N2 // BR,),
    in_specs=[
        pl.BlockSpec((NC, BR, D), lambda i: (0, i, 0)),
        pl.BlockSpec((BR, D), lambda i: (i, 0)),
        pl.BlockSpec((BR, 1), lambda i: (i, 0)),
    ],
    out_specs=[
        pl.BlockSpec((BR, D), lambda i: (i, 0)),
        pl.BlockSpec((BR, D), lambda i: (i, 0)),
    ],
    out_shape=[
        jax.ShapeDtypeStruct((N2, D), jnp.float32),
        jax.ShapeDtypeStruct((N2, D), jnp.float32),
    ],
)


def _fin_tc(p_ref, s_ref, dinv_ref, s_out):
    xn = (p_ref[0] + p_ref[1]) * dinv_ref[...]
    s_out[...] = (s_ref[...] + xn) * (1.0 / (K + 1))


_tc_fin = pl.pallas_call(
    _fin_tc,
    grid=(N2 // BR,),
    in_specs=[
        pl.BlockSpec((NC, BR, D), lambda i: (0, i, 0)),
        pl.BlockSpec((BR, D), lambda i: (i, 0)),
        pl.BlockSpec((BR, 1), lambda i: (i, 0)),
    ],
    out_specs=pl.BlockSpec((BR, D), lambda i: (i, 0)),
    out_shape=jax.ShapeDtypeStruct((N2, D), jnp.float32),
)


def kernel(inputs, edge_index):
    src = edge_index[0].astype(jnp.int32)
    dst = edge_index[1].astype(jnp.int32)
    loop = jnp.arange(N, dtype=jnp.int32)
    pad = jnp.full((E2 - (src.shape[0] + N),), N, jnp.int32)
    src3 = jnp.concatenate([src, loop, pad]).reshape(NW, NCH, CB)
    dst3 = jnp.concatenate([dst, loop, pad]).reshape(NW, NCH, CB)
    xpad = jnp.zeros((N2, D), jnp.float32).at[:N].set(inputs)

    dp = _sc_deg(dst3)
    dinvc, y = _tc_init(dp.reshape(NC, N2, 1), xpad)
    s = xpad
    for k in range(K):
        p = _sc_hop(y, src3, dst3)
        if k < K - 1:
            s, y = _tc_mid(p, s, dinvc)
        else:
            s = _tc_fin(p, s, dinvc)
    return s[:N]


# speedup vs baseline: 1.7832x; 1.6144x over previous
import functools

import jax
import jax.numpy as jnp
from jax import lax
from jax.experimental import pallas as pl
from jax.experimental.pallas import tpu as pltpu
from jax.experimental.pallas import tpu_sc as plsc

N = 10000
D = 128
K = 3
NC = 2
NS = 16
NW = NC * NS
N2 = 10240
RPT = N2 // NS
CB = 128
NCH = 82
NCHH = NCH // 2
EPW = NCH * CB
E2 = EPW * NW
BR = 320


def _mesh():
    return plsc.VectorSubcoreMesh(
        core_axis_name="c", subcore_axis_name="s",
        num_cores=NC, num_subcores=NS)


def _deg_body(dst4, dp_out, ei_v, ones_v, zb_v, db_v, accd):
    cid = lax.axis_index("c")
    sid = lax.axis_index("s")
    wid = sid * NC + cid
    z16 = jnp.zeros((16,), jnp.float32)
    o16 = jnp.ones((16,), jnp.float32)
    for i in range(CB // 16):
        ones_v[pl.ds(i * 16, 16)] = o16

    def fill(i, c):
        zb_v[pl.ds(i * 16, 16)] = z16
        return c

    lax.fori_loop(0, RPT // 16, fill, 0)
    pltpu.sync_copy(zb_v, accd.at[pl.ds(sid * RPT, RPT)])
    plsc.subcore_barrier()

    pltpu.sync_copy(dst4.at[wid], ei_v)

    def step(j, c):
        pltpu.sync_copy(ones_v, accd.at[ei_v.at[j]], add=True)
        return c

    lax.fori_loop(0, NCH, step, 0)
    plsc.subcore_barrier()
    pltpu.sync_copy(accd.at[pl.ds(sid * RPT, RPT)], db_v)
    pltpu.sync_copy(db_v, dp_out.at[cid, pl.ds(sid * RPT, RPT)])


_sc_deg = pl.kernel(
    _deg_body,
    out_type=jax.ShapeDtypeStruct((NC, N2), jnp.float32),
    mesh=_mesh(),
    scratch_types=[
        pltpu.VMEM((NCH, CB), jnp.int32),
        pltpu.VMEM((CB,), jnp.float32),
        pltpu.VMEM((RPT,), jnp.float32),
        pltpu.VMEM((RPT,), jnp.float32),
        pltpu.VMEM_SHARED((N2,), jnp.float32),
    ],
)


def _hop_body(y_hbm, src3, dst3, p_out, src_v, dst_v, rows_v, acc, sem):
    cid = lax.axis_index("c")
    sid = lax.axis_index("s")
    wid = sid * NC + cid
    pltpu.sync_copy(src3.at[wid], src_v)
    pltpu.sync_copy(dst3.at[wid], dst_v)
    z16 = jnp.zeros((16,), jnp.float32)

    def zfill(r, c):
        for col in range(D // 16):
            rows_v[r, pl.ds(col * 16, 16)] = z16
        return c

    lax.fori_loop(0, CB, zfill, 0)
    for b in range(RPT // CB):
        pltpu.sync_copy(rows_v, acc.at[pl.ds(sid * RPT + b * CB, CB)])
    plsc.subcore_barrier()

    def step(j, c):
        pltpu.async_copy(y_hbm.at[src_v.at[j]], rows_v, sem).wait()
        pltpu.sync_copy(rows_v, acc.at[dst_v.at[j]], add=True)
        return c

    lax.fori_loop(0, NCH, step, 0)
    plsc.subcore_barrier()
    for b in range(RPT // CB):
        base = sid * RPT + b * CB
        pltpu.sync_copy(acc.at[pl.ds(base, CB)], rows_v)
        pltpu.sync_copy(rows_v, p_out.at[cid, pl.ds(base, CB)])


_sc_hop = pl.kernel(
    _hop_body,
    out_type=jax.ShapeDtypeStruct((NC, N2, D), jnp.float32),
    mesh=_mesh(),
    scratch_types=[
        pltpu.VMEM((NCH, CB), jnp.int32),
        pltpu.VMEM((NCH, CB), jnp.int32),
        pltpu.VMEM((CB, D), jnp.float32),
        pltpu.VMEM_SHARED((N2, D), jnp.float32),
        pltpu.SemaphoreType.DMA,
    ],
)


def _init_tc(dp_ref, x_ref, dinv_ref, y_ref):
    deg = dp_ref[0] + dp_ref[1]
    dinv = lax.rsqrt(jnp.maximum(deg, 1.0))
    dinv_ref[...] = dinv
    y_ref[...] = x_ref[...] * dinv


_tc_init = pl.pallas_call(
    _init_tc,
    grid=(N2 // BR,),
    in_specs=[
        pl.BlockSpec((NC, BR, 1), lambda i: (0, i, 0)),
        pl.BlockSpec((BR, D), lambda i: (i, 0)),
    ],
    out_specs=[
        pl.BlockSpec((BR, 1), lambda i: (i, 0)),
        pl.BlockSpec((BR, D), lambda i: (i, 0)),
    ],
    out_shape=[
        jax.ShapeDtypeStruct((N2, 1), jnp.float32),
        jax.ShapeDtypeStruct((N2, D), jnp.float32),
    ],
)


def _mid_tc(p_ref, s_ref, dinv_ref, s_out, y_out):
    dinv = dinv_ref[...]
    xn = (p_ref[0] + p_ref[1]) * dinv
    s_out[...] = s_ref[...] + xn
    y_out[...] = xn * dinv


_tc_mid = pl.pallas_call(
    _mid_tc,
    grid=(N2 // BR,),
    in_specs=[
        pl.BlockSpec((NC, BR, D), lambda i: (0, i, 0)),
        pl.BlockSpec((BR, D), lambda i: (i, 0)),
        pl.BlockSpec((BR, 1), lambda i: (i, 0)),
    ],
    out_specs=[
        pl.BlockSpec((BR, D), lambda i: (i, 0)),
        pl.BlockSpec((BR, D), lambda i: (i, 0)),
    ],
    out_shape=[
        jax.ShapeDtypeStruct((N2, D), jnp.float32),
        jax.ShapeDtypeStruct((N2, D), jnp.float32),
    ],
)


def _fin_tc(p_ref, s_ref, dinv_ref, s_out):
    xn = (p_ref[0] + p_ref[1]) * dinv_ref[...]
    s_out[...] = (s_ref[...] + xn) * (1.0 / (K + 1))


_tc_fin = pl.pallas_call(
    _fin_tc,
    grid=(N2 // BR,),
    in_specs=[
        pl.BlockSpec((NC, BR, D), lambda i: (0, i, 0)),
        pl.BlockSpec((BR, D), lambda i: (i, 0)),
        pl.BlockSpec((BR, 1), lambda i: (i, 0)),
    ],
    out_specs=pl.BlockSpec((BR, D), lambda i: (i, 0)),
    out_shape=jax.ShapeDtypeStruct((N2, D), jnp.float32),
)


def kernel(inputs, edge_index):
    src = edge_index[0].astype(jnp.int32)
    dst = edge_index[1].astype(jnp.int32)
    loop = jnp.arange(N, dtype=jnp.int32)
    pad = jnp.full((E2 - (src.shape[0] + N),), N, jnp.int32)
    src3 = jnp.concatenate([src, loop, pad]).reshape(NW, NCH, CB)
    dst3 = jnp.concatenate([dst, loop, pad]).reshape(NW, NCH, CB)
    xpad = jnp.zeros((N2, D), jnp.float32).at[:N].set(inputs)

    dp = _sc_deg(dst3)
    dinvc, y = _tc_init(dp.reshape(NC, N2, 1), xpad)
    s = xpad
    for k in range(K):
        p = _sc_hop(y, src3, dst3)
        if k < K - 1:
            s, y = _tc_mid(p, s, dinvc)
        else:
            s = _tc_fin(p, s, dinvc)
    return s[:N]


# speedup vs baseline: 3.6168x; 2.0283x over previous
import functools

import jax
import jax.numpy as jnp
from jax import lax
from jax.experimental import pallas as pl
from jax.experimental.pallas import tpu as pltpu
from jax.experimental.pallas import tpu_sc as plsc

N = 10000
D = 128
K = 3
NC = 2
NS = 16
NW = NC * NS
N2 = 10240
RPT = N2 // NS
CB = 128
NCH = 82
NCHH = NCH // 2
EPW = NCH * CB
E2 = EPW * NW
BR = 320


def _mesh():
    return plsc.VectorSubcoreMesh(
        core_axis_name="c", subcore_axis_name="s",
        num_cores=NC, num_subcores=NS)


def _deg_body(dst4, dp_out, ei_v, ones_v, zb_v, db_v, accd):
    cid = lax.axis_index("c")
    sid = lax.axis_index("s")
    wid = sid * NC + cid
    z16 = jnp.zeros((16,), jnp.float32)
    o16 = jnp.ones((16,), jnp.float32)
    for i in range(CB // 16):
        ones_v[pl.ds(i * 16, 16)] = o16

    def fill(i, c):
        zb_v[pl.ds(i * 16, 16)] = z16
        return c

    lax.fori_loop(0, RPT // 16, fill, 0)
    pltpu.sync_copy(zb_v, accd.at[pl.ds(sid * RPT, RPT)])
    plsc.subcore_barrier()

    pltpu.sync_copy(dst4.at[wid], ei_v)

    def step(j, c):
        pltpu.sync_copy(ones_v, accd.at[ei_v.at[j]], add=True)
        return c

    lax.fori_loop(0, NCH, step, 0)
    plsc.subcore_barrier()
    pltpu.sync_copy(accd.at[pl.ds(sid * RPT, RPT)], db_v)
    pltpu.sync_copy(db_v, dp_out.at[cid, pl.ds(sid * RPT, RPT)])


_sc_deg = pl.kernel(
    _deg_body,
    out_type=jax.ShapeDtypeStruct((NC, N2), jnp.float32),
    mesh=_mesh(),
    scratch_types=[
        pltpu.VMEM((NCH, CB), jnp.int32),
        pltpu.VMEM((CB,), jnp.float32),
        pltpu.VMEM((RPT,), jnp.float32),
        pltpu.VMEM((RPT,), jnp.float32),
        pltpu.VMEM_SHARED((N2,), jnp.float32),
    ],
)


def _hop_body(y_hbm, src3, dst3, p_out, src_v, dst_v, rows_v, acc, sem):
    cid = lax.axis_index("c")
    sid = lax.axis_index("s")
    wid = sid * NC + cid
    pltpu.sync_copy(src3.at[wid], src_v)
    pltpu.sync_copy(dst3.at[wid], dst_v)
    z16 = jnp.zeros((16,), jnp.float32)

    def zfill(r, c):
        for col in range(D // 16):
            rows_v[r, pl.ds(col * 16, 16)] = z16
        return c

    lax.fori_loop(0, CB, zfill, 0)
    for b in range(RPT // CB):
        pltpu.sync_copy(rows_v, acc.at[pl.ds(sid * RPT + b * CB, CB)])
    plsc.subcore_barrier()

    def step(j, c):
        pltpu.async_copy(y_hbm.at[src_v.at[j]], rows_v, sem).wait()
        pltpu.sync_copy(rows_v, acc.at[dst_v.at[j]], add=True)
        return c

    lax.fori_loop(0, NCH, step, 0)
    plsc.subcore_barrier()
    for b in range(RPT // CB):
        base = sid * RPT + b * CB
        pltpu.sync_copy(acc.at[pl.ds(base, CB)], rows_v)
        pltpu.sync_copy(rows_v, p_out.at[cid, pl.ds(base, CB)])


_sc_hop = pl.kernel(
    _hop_body,
    out_type=jax.ShapeDtypeStruct((NC, N2, D), jnp.float32),
    mesh=_mesh(),
    scratch_types=[
        pltpu.VMEM((NCH, CB), jnp.int32),
        pltpu.VMEM((NCH, CB), jnp.int32),
        pltpu.VMEM((CB, D), jnp.float32),
        pltpu.VMEM_SHARED((N2, D), jnp.float32),
        pltpu.SemaphoreType.DMA,
    ],
)


def _init_tc(dp_ref, x_ref, dinv_ref, y_ref):
    deg = dp_ref[0] + dp_ref[1]
    dinv = lax.rsqrt(jnp.maximum(deg, 1.0))
    dinv_ref[...] = dinv
    y_ref[...] = x_ref[...] * dinv


_tc_init = pl.pallas_call(
    _init_tc,
    grid=(N2 // BR,),
    in_specs=[
        pl.BlockSpec((NC, BR, 1), lambda i: (0, i, 0)),
        pl.BlockSpec((BR, D), lambda i: (i, 0)),
    ],
    out_specs=[
        pl.BlockSpec((BR, 1), lambda i: (i, 0)),
        pl.BlockSpec((BR, D), lambda i: (i, 0)),
    ],
    out_shape=[
        jax.ShapeDtypeStruct((N2, 1), jnp.float32),
        jax.ShapeDtypeStruct((N2, D), jnp.float32),
    ],
)


def _mid_tc(p_ref, s_ref, dinv_ref, s_out, y_out):
    dinv = dinv_ref[...]
    xn = (p_ref[0] + p_ref[1]) * dinv
    s_out[...] = s_ref[...] + xn
    y_out[...] = xn * dinv


_tc_mid = pl.pallas_call(
    _mid_tc,
    grid=(N2 // BR,),
    in_specs=[
        pl.BlockSpec((NC, BR, D), lambda i: (0, i, 0)),
        pl.BlockSpec((BR, D), lambda i: (i, 0)),
        pl.BlockSpec((BR, 1), lambda i: (i, 0)),
    ],
    out_specs=[
        pl.BlockSpec((BR, D), lambda i: (i, 0)),
        pl.BlockSpec((BR, D), lambda i: (i, 0)),
    ],
    out_shape=[
        jax.ShapeDtypeStruct((N2, D), jnp.float32),
        jax.ShapeDtypeStruct((N2, D), jnp.float32),
    ],
)


def _fin_tc(p_ref, s_ref, dinv_ref, s_out):
    xn = (p_ref[0] + p_ref[1]) * dinv_ref[...]
    s_out[...] = (s_ref[...] + xn) * (1.0 / (K + 1))


_tc_fin = pl.pallas_call(
    _fin_tc,
    grid=(N2 // BR,),
    in_specs=[
        pl.BlockSpec((NC, BR, D), lambda i: (0, i, 0)),
        pl.BlockSpec((BR, D), lambda i: (i, 0)),
        pl.BlockSpec((BR, 1), lambda i: (i, 0)),
    ],
    out_specs=pl.BlockSpec((BR, D), lambda i: (i, 0)),
    out_shape=jax.ShapeDtypeStruct((N2, D), jnp.float32),
)


def kernel(inputs, edge_index):
    src = edge_index[0].astype(jnp.int32)
    dst = edge_index[1].astype(jnp.int32)
    loop = jnp.arange(N, dtype=jnp.int32)
    npad = E2 - (src.shape[0] + N)
    pad = N + (jnp.arange(npad, dtype=jnp.int32) % (N2 - N))
    src3 = jnp.concatenate([src, loop, pad]).reshape(NW, NCH, CB)
    dst3 = jnp.concatenate([dst, loop, pad]).reshape(NW, NCH, CB)
    xpad = jnp.zeros((N2, D), jnp.float32).at[:N].set(inputs)

    dp = _sc_deg(dst3)
    dinvc, y = _tc_init(dp.reshape(NC, N2, 1), xpad)
    s = xpad
    for k in range(K):
        p = _sc_hop(y, src3, dst3)
        if k < K - 1:
            s, y = _tc_mid(p, s, dinvc)
        else:
            s = _tc_fin(p, s, dinvc)
    return s[:N]


# speedup vs baseline: 4.4624x; 1.2338x over previous
import functools

import jax
import jax.numpy as jnp
from jax import lax
from jax.experimental import pallas as pl
from jax.experimental.pallas import tpu as pltpu
from jax.experimental.pallas import tpu_sc as plsc

N = 10000
D = 128
K = 3
NC = 2
NS = 16
NW = NC * NS
N2 = 10240
RPT = N2 // NS
CB = 128
NCH = 84
NCHH = NCH // 2
EPW = NCH * CB
E2 = EPW * NW
BR = 320


def _mesh():
    return plsc.VectorSubcoreMesh(
        core_axis_name="c", subcore_axis_name="s",
        num_cores=NC, num_subcores=NS)


def _deg_body(dst4, dp_out, ei_v, ones_v, zb_v, db_v, accd):
    cid = lax.axis_index("c")
    sid = lax.axis_index("s")
    wid = sid * NC + cid
    z16 = jnp.zeros((16,), jnp.float32)
    o16 = jnp.ones((16,), jnp.float32)
    for i in range(CB // 16):
        ones_v[pl.ds(i * 16, 16)] = o16

    def fill(i, c):
        zb_v[pl.ds(i * 16, 16)] = z16
        return c

    lax.fori_loop(0, RPT // 16, fill, 0)
    pltpu.sync_copy(zb_v, accd.at[pl.ds(sid * RPT, RPT)])
    plsc.subcore_barrier()

    for h in range(2):
        pltpu.sync_copy(dst4.at[wid, h], ei_v)

        def step(j, c):
            pltpu.sync_copy(ones_v, accd.at[ei_v.at[j]], add=True)
            return c

        lax.fori_loop(0, NCHH, step, 0)
    plsc.subcore_barrier()
    pltpu.sync_copy(accd.at[pl.ds(sid * RPT, RPT)], db_v)
    pltpu.sync_copy(db_v, dp_out.at[cid, pl.ds(sid * RPT, RPT)])


_sc_deg = pl.kernel(
    _deg_body,
    out_type=jax.ShapeDtypeStruct((NC, N2), jnp.float32),
    mesh=_mesh(),
    scratch_types=[
        pltpu.VMEM((NCHH, CB), jnp.int32),
        pltpu.VMEM((CB,), jnp.float32),
        pltpu.VMEM((RPT,), jnp.float32),
        pltpu.VMEM((RPT,), jnp.float32),
        pltpu.VMEM_SHARED((N2,), jnp.float32),
    ],
)


def _hop_body(y_hbm, src4, dst4, p_out, src_v, dst_v, rows0, rows1, acc,
              gs0, gs1):
    cid = lax.axis_index("c")
    sid = lax.axis_index("s")
    wid = sid * NC + cid
    z16 = jnp.zeros((16,), jnp.float32)

    def zfill(r, c):
        for col in range(D // 16):
            rows0[r, pl.ds(col * 16, 16)] = z16
        return c

    lax.fori_loop(0, CB, zfill, 0)
    for b in range(RPT // CB):
        pltpu.sync_copy(rows0, acc.at[pl.ds(sid * RPT + b * CB, CB)])
    plsc.subcore_barrier()

    for h in range(2):
        pltpu.sync_copy(src4.at[wid, h], src_v)
        pltpu.sync_copy(dst4.at[wid, h], dst_v)
        pltpu.async_copy(y_hbm.at[src_v.at[0]], rows0, gs0)

        def pair(t, c):
            j0 = 2 * t
            pltpu.make_async_copy(
                y_hbm.at[src_v.at[j0]], rows0, gs0).wait()
            pltpu.async_copy(y_hbm.at[src_v.at[j0 + 1]], rows1, gs1)
            pltpu.sync_copy(rows0, acc.at[dst_v.at[j0]], add=True)
            pltpu.make_async_copy(
                y_hbm.at[src_v.at[j0 + 1]], rows1, gs1).wait()

            @pl.when(t < NCHH // 2 - 1)
            def _():
                pltpu.async_copy(y_hbm.at[src_v.at[j0 + 2]], rows0, gs0)

            pltpu.sync_copy(rows1, acc.at[dst_v.at[j0 + 1]], add=True)
            return c

        lax.fori_loop(0, NCHH // 2, pair, 0)
    plsc.subcore_barrier()
    for b in range(RPT // CB):
        base = sid * RPT + b * CB
        pltpu.sync_copy(acc.at[pl.ds(base, CB)], rows0)
        pltpu.sync_copy(rows0, p_out.at[cid, pl.ds(base, CB)])


_sc_hop = pl.kernel(
    _hop_body,
    out_type=jax.ShapeDtypeStruct((NC, N2, D), jnp.float32),
    mesh=_mesh(),
    scratch_types=[
        pltpu.VMEM((NCHH, CB), jnp.int32),
        pltpu.VMEM((NCHH, CB), jnp.int32),
        pltpu.VMEM((CB, D), jnp.float32),
        pltpu.VMEM((CB, D), jnp.float32),
        pltpu.VMEM_SHARED((N2, D), jnp.float32),
        pltpu.SemaphoreType.DMA,
        pltpu.SemaphoreType.DMA,
    ],
)


def _init_tc(dp_ref, x_ref, dinv_ref, y_ref):
    deg = dp_ref[0] + dp_ref[1]
    dinv = lax.rsqrt(jnp.maximum(deg, 1.0))
    dinv_ref[...] = dinv
    y_ref[...] = x_ref[...] * dinv


_tc_init = pl.pallas_call(
    _init_tc,
    grid=(N2 // BR,),
    in_specs=[
        pl.BlockSpec((NC, BR, 1), lambda i: (0, i, 0)),
        pl.BlockSpec((BR, D), lambda i: (i, 0)),
    ],
    out_specs=[
        pl.BlockSpec((BR, 1), lambda i: (i, 0)),
        pl.BlockSpec((BR, D), lambda i: (i, 0)),
    ],
    out_shape=[
        jax.ShapeDtypeStruct((N2, 1), jnp.float32),
        jax.ShapeDtypeStruct((N2, D), jnp.float32),
    ],
)


def _mid_tc(p_ref, s_ref, dinv_ref, s_out, y_out):
    dinv = dinv_ref[...]
    xn = (p_ref[0] + p_ref[1]) * dinv
    s_out[...] = s_ref[...] + xn
    y_out[...] = xn * dinv


_tc_mid = pl.pallas_call(
    _mid_tc,
    grid=(N2 // BR,),
    in_specs=[
        pl.BlockSpec((NC, BR, D), lambda i: (0, i, 0)),
        pl.BlockSpec((BR, D), lambda i: (i, 0)),
        pl.BlockSpec((BR, 1), lambda i: (i, 0)),
    ],
    out_specs=[
        pl.BlockSpec((BR, D), lambda i: (i, 0)),
        pl.BlockSpec((BR, D), lambda i: (i, 0)),
    ],
    out_shape=[
        jax.ShapeDtypeStruct((N2, D), jnp.float32),
        jax.ShapeDtypeStruct((N2, D), jnp.float32),
    ],
)


def _fin_tc(p_ref, s_ref, dinv_ref, s_out):
    xn = (p_ref[0] + p_ref[1]) * dinv_ref[...]
    s_out[...] = (s_ref[...] + xn) * (1.0 / (K + 1))


_tc_fin = pl.pallas_call(
    _fin_tc,
    grid=(N2 // BR,),
    in_specs=[
        pl.BlockSpec((NC, BR, D), lambda i: (0, i, 0)),
        pl.BlockSpec((BR, D), lambda i: (i, 0)),
        pl.BlockSpec((BR, 1), lambda i: (i, 0)),
    ],
    out_specs=pl.BlockSpec((BR, D), lambda i: (i, 0)),
    out_shape=jax.ShapeDtypeStruct((N2, D), jnp.float32),
)


def kernel(inputs, edge_index):
    src = edge_index[0].astype(jnp.int32)
    dst = edge_index[1].astype(jnp.int32)
    loop = jnp.arange(N, dtype=jnp.int32)
    npad = E2 - (src.shape[0] + N)
    pad = N + (jnp.arange(npad, dtype=jnp.int32) % (N2 - N))
    src3 = jnp.concatenate([src, loop, pad]).reshape(NW, 2, NCHH, CB)
    dst3 = jnp.concatenate([dst, loop, pad]).reshape(NW, 2, NCHH, CB)
    xpad = jnp.zeros((N2, D), jnp.float32).at[:N].set(inputs)

    dp = _sc_deg(dst3)
    dinvc, y = _tc_init(dp.reshape(NC, N2, 1), xpad)
    s = xpad
    for k in range(K):
        p = _sc_hop(y, src3, dst3)
        if k < K - 1:
            s, y = _tc_mid(p, s, dinvc)
        else:
            s = _tc_fin(p, s, dinvc)
    return s[:N]
